# baseline (speedup 1.0000x reference)
import functools

import jax
import jax.numpy as jnp
from jax.experimental import pallas as pl

B, N, K = 2, 2048, 16
D = 1024
H = 16
HD = 64
DIM = H * HD

ROW_BLK = 512
I_BLK = 128
SCALE = HD ** -0.5
LOG2E = 1.4426950408889634


def _kv_proj_kernel(ctx_ref, wk_ref, wv_ref, k_ref, v_ref):
    c = ctx_ref[...]
    k_ref[...] = jnp.dot(c, wk_ref[...], preferred_element_type=jnp.float32)
    v_ref[...] = jnp.dot(c, wv_ref[...], preferred_element_type=jnp.float32)


def _attn_kernel(x_ref, idx_ref, k_ref, v_ref, wq_ref, bq_ref, wo_ref,
                 bo_ref, o_ref):
    xb = x_ref[0]
    q = jnp.dot(xb, wq_ref[...], preferred_element_type=jnp.float32)
    q = (q + bq_ref[...]) * (SCALE * LOG2E)

    idx = idx_ref[0]
    jj = jax.lax.broadcasted_iota(jnp.int32, (I_BLK, N), 1)
    counts = jnp.zeros((I_BLK, N), jnp.float32)
    for kk in range(K):
        counts = counts + (idx[:, kk:kk + 1] == jj).astype(jnp.float32)
    lc = jnp.log2(counts)

    parts = []
    for h in range(H):
        qh = q[:, h * HD:(h + 1) * HD]
        kh = k_ref[0, :, h * HD:(h + 1) * HD]
        t = jax.lax.dot_general(qh, kh, (((1,), (1,)), ((), ())),
                                preferred_element_type=jnp.float32)
        t = t + lc
        m = jnp.max(t, axis=1, keepdims=True)
        p = jnp.exp2(t - m)
        denom = jnp.sum(p, axis=1, keepdims=True)
        vh = v_ref[0, :, h * HD:(h + 1) * HD]
        oh = jax.lax.dot_general(p, vh, (((1,), (0,)), ((), ())),
                                 preferred_element_type=jnp.float32)
        parts.append(oh * (1.0 / denom))
    attn = jnp.concatenate(parts, axis=1)
    o_ref[0] = (jnp.dot(attn, wo_ref[...], preferred_element_type=jnp.float32)
                + bo_ref[...])


def kernel(x, context, index_pairs, Wq, bq, Wk, Wv, Wo, bo):
    ctx2 = context.reshape(B * N, D)
    idx = index_pairs.astype(jnp.int32)

    kv = pl.pallas_call(
        _kv_proj_kernel,
        grid=(B * N // ROW_BLK,),
        in_specs=[
            pl.BlockSpec((ROW_BLK, D), lambda r: (r, 0)),
            pl.BlockSpec((D, DIM), lambda r: (0, 0)),
            pl.BlockSpec((D, DIM), lambda r: (0, 0)),
        ],
        out_specs=[
            pl.BlockSpec((ROW_BLK, DIM), lambda r: (r, 0)),
            pl.BlockSpec((ROW_BLK, DIM), lambda r: (r, 0)),
        ],
        out_shape=[
            jax.ShapeDtypeStruct((B * N, DIM), jnp.float32),
            jax.ShapeDtypeStruct((B * N, DIM), jnp.float32),
        ],
    )(ctx2, Wk, Wv)
    k3 = kv[0].reshape(B, N, DIM)
    v3 = kv[1].reshape(B, N, DIM)

    out = pl.pallas_call(
        _attn_kernel,
        grid=(B, N // I_BLK),
        in_specs=[
            pl.BlockSpec((1, I_BLK, D), lambda b, i: (b, i, 0)),
            pl.BlockSpec((1, I_BLK, K), lambda b, i: (b, i, 0)),
            pl.BlockSpec((1, N, DIM), lambda b, i: (b, 0, 0)),
            pl.BlockSpec((1, N, DIM), lambda b, i: (b, 0, 0)),
            pl.BlockSpec((D, DIM), lambda b, i: (0, 0)),
            pl.BlockSpec((1, DIM), lambda b, i: (0, 0)),
            pl.BlockSpec((DIM, D), lambda b, i: (0, 0)),
            pl.BlockSpec((1, D), lambda b, i: (0, 0)),
        ],
        out_specs=pl.BlockSpec((1, I_BLK, D), lambda b, i: (b, i, 0)),
        out_shape=jax.ShapeDtypeStruct((B, N, D), jnp.float32),
    )(x, idx, k3, v3, Wq, bq.reshape(1, DIM), Wo, bo.reshape(1, D))
    return out


# speedup vs baseline: 1.3002x; 1.3002x over previous
import jax
import jax.numpy as jnp
from jax.experimental import pallas as pl

B, N, K = 2, 2048, 16
D = 1024
H = 16
HD = 64
DIM = H * HD

ROW_BLK = 512
I_BLK = 256
SCALE = HD ** -0.5
LOG2E = 1.4426950408889634


def _qkv_proj_kernel(x_ref, ctx_ref, wq_ref, bq_ref, wk_ref, wv_ref,
                     q_ref, k_ref, v_ref):
    xb = x_ref[...]
    c = ctx_ref[...]
    q = jnp.dot(xb, wq_ref[...], preferred_element_type=jnp.float32)
    q_ref[...] = (q + bq_ref[...]) * (SCALE * LOG2E)
    k_ref[...] = jnp.dot(c, wk_ref[...], preferred_element_type=jnp.float32)
    v_ref[...] = jnp.dot(c, wv_ref[...], preferred_element_type=jnp.float32)


def _attn_kernel(q_ref, idx_ref, k_ref, v_ref, wo_ref, bo_ref, o_ref):
    q = q_ref[0]
    idx = idx_ref[0]
    jj = jax.lax.broadcasted_iota(jnp.int32, (I_BLK, N), 1)
    counts = jnp.zeros((I_BLK, N), jnp.float32)
    for kk in range(K):
        counts = counts + (idx[:, kk:kk + 1] == jj).astype(jnp.float32)
    lc = jnp.log2(counts)

    parts = []
    for h in range(H):
        qh = q[:, h * HD:(h + 1) * HD]
        kh = k_ref[0, :, h * HD:(h + 1) * HD]
        t = jax.lax.dot_general(qh, kh, (((1,), (1,)), ((), ())),
                                preferred_element_type=jnp.float32)
        t = t + lc
        m = jnp.max(t, axis=1, keepdims=True)
        p = jnp.exp2(t - m)
        denom = jnp.sum(p, axis=1, keepdims=True)
        vh = v_ref[0, :, h * HD:(h + 1) * HD]
        oh = jax.lax.dot_general(p, vh, (((1,), (0,)), ((), ())),
                                 preferred_element_type=jnp.float32)
        parts.append(oh * (1.0 / denom))
    attn = jnp.concatenate(parts, axis=1)
    o_ref[0] = (jnp.dot(attn, wo_ref[...], preferred_element_type=jnp.float32)
                + bo_ref[...])


def kernel(x, context, index_pairs, Wq, bq, Wk, Wv, Wo, bo):
    x2 = x.reshape(B * N, D)
    ctx2 = context.reshape(B * N, D)
    idx = index_pairs.astype(jnp.int32)

    qkv = pl.pallas_call(
        _qkv_proj_kernel,
        grid=(B * N // ROW_BLK,),
        in_specs=[
            pl.BlockSpec((ROW_BLK, D), lambda r: (r, 0)),
            pl.BlockSpec((ROW_BLK, D), lambda r: (r, 0)),
            pl.BlockSpec((D, DIM), lambda r: (0, 0)),
            pl.BlockSpec((1, DIM), lambda r: (0, 0)),
            pl.BlockSpec((D, DIM), lambda r: (0, 0)),
            pl.BlockSpec((D, DIM), lambda r: (0, 0)),
        ],
        out_specs=[
            pl.BlockSpec((ROW_BLK, DIM), lambda r: (r, 0)),
            pl.BlockSpec((ROW_BLK, DIM), lambda r: (r, 0)),
            pl.BlockSpec((ROW_BLK, DIM), lambda r: (r, 0)),
        ],
        out_shape=[
            jax.ShapeDtypeStruct((B * N, DIM), jnp.float32),
            jax.ShapeDtypeStruct((B * N, DIM), jnp.float32),
            jax.ShapeDtypeStruct((B * N, DIM), jnp.float32),
        ],
    )(x2, ctx2, Wq, bq.reshape(1, DIM), Wk, Wv)
    q3 = qkv[0].reshape(B, N, DIM)
    k3 = qkv[1].reshape(B, N, DIM)
    v3 = qkv[2].reshape(B, N, DIM)

    out = pl.pallas_call(
        _attn_kernel,
        grid=(B, N // I_BLK),
        in_specs=[
            pl.BlockSpec((1, I_BLK, DIM), lambda b, i: (b, i, 0)),
            pl.BlockSpec((1, I_BLK, K), lambda b, i: (b, i, 0)),
            pl.BlockSpec((1, N, DIM), lambda b, i: (b, 0, 0)),
            pl.BlockSpec((1, N, DIM), lambda b, i: (b, 0, 0)),
            pl.BlockSpec((DIM, D), lambda b, i: (0, 0)),
            pl.BlockSpec((1, D), lambda b, i: (0, 0)),
        ],
        out_specs=pl.BlockSpec((1, I_BLK, D), lambda b, i: (b, i, 0)),
        out_shape=jax.ShapeDtypeStruct((B, N, D), jnp.float32),
    )(q3, idx, k3, v3, Wo, bo.reshape(1, D))
    return out
